# Initial kernel scaffold; baseline (speedup 1.0000x reference)
import jax
import jax.numpy as jnp
from jax.experimental import pallas as pl


def kernel(x, label):
    raise NotImplementedError("write your pallas kernel here")


# speedup vs baseline: 1.4469x; 1.4469x over previous
import functools

import jax
import jax.numpy as jnp
from jax import lax
from jax.experimental import pallas as pl
from jax.experimental.pallas import tpu as pltpu
from jax.experimental.pallas import tpu_sc as plsc

N_ROWS = 16384
N_COLS = 1000
LANES = 16
N_WORKERS = 32
ROWS_PER_WORKER = N_ROWS // N_WORKERS
G = 16
N_PAIRS = ROWS_PER_WORKER // (2 * G)
NEG = -10.0 ** 10
MARGIN_FY = 0.8
MARGIN_FNY = 0.3
N_CHUNKS = N_COLS // LANES


def _sc_body(x_hbm, lbl_hbm, out_hbm, xbuf, lblbuf, ostage, sem0, sem1):
    wid = lax.axis_index("c") * 16 + lax.axis_index("s")
    row0 = wid * ROWS_PER_WORKER

    pltpu.sync_copy(lbl_hbm.at[pl.ds(row0, ROWS_PER_WORKER)], lblbuf)

    def dma(g, slot, sem):
        src = x_hbm.at[pl.ds((row0 + g * G) * N_COLS, G * N_COLS)]
        dst = xbuf.at[pl.ds(slot * G * N_COLS, G * N_COLS)]
        return pltpu.make_async_copy(src, dst, sem)

    dma(0, 0, sem0).start()
    dma(1, 1, sem1).start()

    lane = lax.iota(jnp.int32, LANES)

    def process(g, slot, sem, acc):
        dma(g, slot, sem).wait()
        labels = lblbuf[pl.ds(g * G, G)]
        base = slot * G * N_COLS
        idx = lane * N_COLS + labels + base
        fy = plsc.load_gather(xbuf, [idx])
        plsc.store_scatter(xbuf, [idx], jnp.full((LANES,), NEG, jnp.float32))

        def row_max(r, m_vec):
            rbase = base + r * N_COLS
            chunks = [xbuf[pl.ds(rbase + LANES * j, LANES)]
                      for j in range(N_CHUNKS)]
            chunks.append(xbuf[pl.ds(rbase + (N_COLS - LANES), LANES)])
            while len(chunks) > 1:
                nxt = [jnp.maximum(chunks[i], chunks[i + 1])
                       for i in range(0, len(chunks) - 1, 2)]
                if len(chunks) % 2:
                    nxt.append(chunks[-1])
                chunks = nxt
            m_r = jnp.max(chunks[0])
            return jnp.where(lane == r, m_r, m_vec)

        m = lax.fori_loop(0, G, row_max, jnp.full((LANES,), NEG, jnp.float32))
        l = jnp.maximum(MARGIN_FY - fy, 0.0) + jnp.maximum(MARGIN_FNY + m, 0.0)
        return acc + l

    def pair(g2, acc):
        g0 = 2 * g2
        acc = process(g0, 0, sem0, acc)

        @pl.when(g2 < N_PAIRS - 1)
        def _():
            dma(g0 + 2, 0, sem0).start()

        acc = process(g0 + 1, 1, sem1, acc)

        @pl.when(g2 < N_PAIRS - 1)
        def _():
            dma(g0 + 3, 1, sem1).start()

        return acc

    acc = lax.fori_loop(0, N_PAIRS, pair, jnp.zeros((LANES,), jnp.float32))
    ostage[...] = acc * (1.0 / N_ROWS)
    pltpu.sync_copy(ostage, out_hbm.at[wid])


_sc_loss = functools.partial(
    pl.kernel,
    out_type=jax.ShapeDtypeStruct((N_WORKERS, LANES), jnp.float32),
    mesh=plsc.VectorSubcoreMesh(core_axis_name="c", subcore_axis_name="s"),
    compiler_params=pltpu.CompilerParams(needs_layout_passes=False),
    scratch_types=[
        pltpu.VMEM((2 * G * N_COLS,), jnp.float32),
        pltpu.VMEM((ROWS_PER_WORKER,), jnp.int32),
        pltpu.VMEM((LANES,), jnp.float32),
        pltpu.SemaphoreType.DMA,
        pltpu.SemaphoreType.DMA,
    ],
)(_sc_body)


def kernel(x, label):
    parts = _sc_loss(x.reshape(-1), label.astype(jnp.int32))
    return jnp.sum(parts)


# speedup vs baseline: 2.3757x; 1.6419x over previous
import functools

import jax
import jax.numpy as jnp
from jax import lax
from jax.experimental import pallas as pl
from jax.experimental.pallas import tpu as pltpu
from jax.experimental.pallas import tpu_sc as plsc

N_ROWS = 16384
N_COLS = 1000
LANES = 16
N_WORKERS = 32
ROWS_PER_WORKER = N_ROWS // N_WORKERS
G = 16
N_PAIRS = ROWS_PER_WORKER // (2 * G)
NEG = -10.0 ** 10
MARGIN_FY = 0.8
MARGIN_FNY = 0.3
N_CHUNKS = N_COLS // LANES


def _sc_body(x_hbm, lbl_hbm, out_hbm, xbuf, lblbuf, ostage, sem0, sem1):
    wid = lax.axis_index("c") * 16 + lax.axis_index("s")
    row0 = wid * ROWS_PER_WORKER

    pltpu.sync_copy(lbl_hbm.at[pl.ds(row0, ROWS_PER_WORKER)], lblbuf)

    def dma(g, slot, sem):
        src = x_hbm.at[pl.ds(row0 + g * G, G), :]
        dst = xbuf.at[pl.ds(slot * G, G), :]
        return pltpu.make_async_copy(src, dst, sem)

    dma(0, 0, sem0).start()
    dma(1, 1, sem1).start()

    lane = lax.iota(jnp.int32, LANES)

    def process(g, slot, sem, acc):
        dma(g, slot, sem).wait()
        labels = lblbuf[pl.ds(g * G, G)]
        rowidx = slot * G + lane
        fy = plsc.load_gather(xbuf, [rowidx, labels])
        plsc.store_scatter(xbuf, [rowidx, labels],
                           jnp.full((LANES,), NEG, jnp.float32))

        def row_max(r, m_vec):
            row = slot * G + r
            chunks = [xbuf[row, pl.ds(LANES * j, LANES)]
                      for j in range(N_CHUNKS)]
            chunks.append(xbuf[row, pl.ds(N_COLS - LANES, LANES)])
            while len(chunks) > 1:
                nxt = [jnp.maximum(chunks[i], chunks[i + 1])
                       for i in range(0, len(chunks) - 1, 2)]
                if len(chunks) % 2:
                    nxt.append(chunks[-1])
                chunks = nxt
            m_r = jnp.max(chunks[0])
            return jnp.where(lane == r, m_r, m_vec)

        m = lax.fori_loop(0, G, row_max, jnp.full((LANES,), NEG, jnp.float32))
        l = jnp.maximum(MARGIN_FY - fy, 0.0) + jnp.maximum(MARGIN_FNY + m, 0.0)
        return acc + l

    def pair(g2, acc):
        g0 = 2 * g2
        acc = process(g0, 0, sem0, acc)

        @pl.when(g2 < N_PAIRS - 1)
        def _():
            dma(g0 + 2, 0, sem0).start()

        acc = process(g0 + 1, 1, sem1, acc)

        @pl.when(g2 < N_PAIRS - 1)
        def _():
            dma(g0 + 3, 1, sem1).start()

        return acc

    acc = lax.fori_loop(0, N_PAIRS, pair, jnp.zeros((LANES,), jnp.float32))
    ostage[...] = acc * (1.0 / N_ROWS)
    pltpu.sync_copy(ostage, out_hbm.at[wid])


_sc_loss = functools.partial(
    pl.kernel,
    out_type=jax.ShapeDtypeStruct((N_WORKERS, LANES), jnp.float32),
    mesh=plsc.VectorSubcoreMesh(core_axis_name="c", subcore_axis_name="s"),
    compiler_params=pltpu.CompilerParams(needs_layout_passes=False),
    scratch_types=[
        pltpu.VMEM((2 * G, N_COLS), jnp.float32),
        pltpu.VMEM((ROWS_PER_WORKER,), jnp.int32),
        pltpu.VMEM((LANES,), jnp.float32),
        pltpu.SemaphoreType.DMA,
        pltpu.SemaphoreType.DMA,
    ],
)(_sc_body)


def kernel(x, label):
    parts = _sc_loss(x, label.astype(jnp.int32))
    return jnp.sum(parts)
